# Initial kernel scaffold; baseline (speedup 1.0000x reference)
import jax
import jax.numpy as jnp
from jax.experimental import pallas as pl


def kernel(h, edge_index, Wq, Wk, Wv, swg_W1, swg_b1, swg_W2, swg_b2, WO, bn1_g, bn1_b, ffn_W1, ffn_b1, ffn_W2, ffn_b2, bn2_g, bn2_b):
    raise NotImplementedError("write your pallas kernel here")


# speedup vs baseline: 11.8562x; 11.8562x over previous
import jax
import jax.numpy as jnp
from jax.experimental import pallas as pl
from jax.experimental.pallas import tpu as pltpu

N = 4096
H = 8
DH = 64
NS = 3
TOPK = 20
RB = 256
MB = 512


def _rowsum_kernel(a_ref, d_ref):
    d_ref[...] = jnp.sum(a_ref[...], axis=1).reshape(1, RB)


def _rowsum(a):
    return pl.pallas_call(
        _rowsum_kernel,
        grid=(N // RB,),
        in_specs=[pl.BlockSpec((RB, N), lambda i: (i, 0))],
        out_specs=pl.BlockSpec((1, RB), lambda i: (0, i)),
        out_shape=jax.ShapeDtypeStruct((1, N), jnp.float32),
    )(a)


def _norm_kernel(a_ref, d_ref, m_ref):
    i = pl.program_id(0)
    d = d_ref[0, :]
    dinv = jnp.where(d > 0, d ** -0.5, 0.0)
    rows = d_ref[0, pl.ds(i * RB, RB)]
    rinv = jnp.where(rows > 0, rows ** -0.5, 0.0)
    m_ref[...] = a_ref[...] * rinv[:, None] * dinv[None, :]


def _normalize(a, d):
    return pl.pallas_call(
        _norm_kernel,
        grid=(N // RB,),
        in_specs=[pl.BlockSpec((RB, N), lambda i: (i, 0)),
                  pl.BlockSpec((1, N), lambda i: (0, 0))],
        out_specs=pl.BlockSpec((RB, N), lambda i: (i, 0)),
        out_shape=jax.ShapeDtypeStruct((N, N), jnp.float32),
    )(a, d)


def _mm_kernel(a_ref, b_ref, o_ref):
    @pl.when(pl.program_id(2) == 0)
    def _():
        o_ref[...] = jnp.zeros_like(o_ref)
    o_ref[...] += jnp.dot(a_ref[...], b_ref[...],
                          preferred_element_type=jnp.float32)


def _matmul(a, b):
    g = N // MB
    return pl.pallas_call(
        _mm_kernel,
        grid=(g, g, g),
        in_specs=[pl.BlockSpec((MB, MB), lambda i, j, k: (i, k)),
                  pl.BlockSpec((MB, MB), lambda i, j, k: (k, j))],
        out_specs=pl.BlockSpec((MB, MB), lambda i, j, k: (i, j)),
        out_shape=jax.ShapeDtypeStruct((N, N), jnp.float32),
        compiler_params=pltpu.CompilerParams(
            dimension_semantics=("parallel", "parallel", "arbitrary")),
    )(a, b)


def _topk_kernel(r_ref, m_ref):
    i = pl.program_id(0)
    x = r_ref[...]
    col = jax.lax.broadcasted_iota(jnp.int32, (RB, N), 1)
    rowg = jax.lax.broadcasted_iota(jnp.int32, (RB, N), 0) + i * RB
    x = jnp.where(col == rowg, 0.0, x)
    mask = jnp.zeros_like(x)
    for _ in range(TOPK):
        mx = jnp.max(x, axis=1, keepdims=True)
        cand = jnp.where(x == mx, col, N)
        jmin = jnp.min(cand, axis=1, keepdims=True)
        sel = col == jmin
        mask = jnp.where(sel, 1.0, mask)
        x = jnp.where(sel, -1.0, x)
    m_ref[...] = mask


def _topk_mask(r):
    return pl.pallas_call(
        _topk_kernel,
        grid=(N // RB,),
        in_specs=[pl.BlockSpec((RB, N), lambda i: (i, 0))],
        out_specs=pl.BlockSpec((RB, N), lambda i: (i, 0)),
        out_shape=jax.ShapeDtypeStruct((N, N), jnp.float32),
    )(r)


def _qkv_kernel(h_ref, wq_ref, wk_ref, wv_ref, q_ref, k_ref, v_ref):
    hb = h_ref[...]
    dn = (((1,), (1,)), ((), ()))
    for s in range(NS):
        q_ref[s] = jax.lax.dot_general(hb, wq_ref[s], dn,
                                       preferred_element_type=jnp.float32)
        k_ref[s] = jax.lax.dot_general(hb, wk_ref[s], dn,
                                       preferred_element_type=jnp.float32)
        v_ref[s] = jax.lax.dot_general(hb, wv_ref[s], dn,
                                       preferred_element_type=jnp.float32)


def _qkv(h, wq, wk, wv):
    wspec = pl.BlockSpec((NS, 512, 512), lambda i: (0, 0, 0))
    ospec = pl.BlockSpec((NS, RB, 512), lambda i: (0, i, 0))
    osh = jax.ShapeDtypeStruct((NS, N, 512), jnp.float32)
    return pl.pallas_call(
        _qkv_kernel,
        grid=(N // RB,),
        in_specs=[pl.BlockSpec((RB, 512), lambda i: (i, 0)), wspec, wspec,
                  wspec],
        out_specs=[ospec, ospec, ospec],
        out_shape=[osh, osh, osh],
    )(h, wq, wk, wv)


def _attn_kernel(k_ref, q_ref, v_ref, b_ref, wv_ref, z_ref):
    @pl.when(pl.program_id(1) == 0)
    def _():
        wv_ref[...] = jnp.zeros_like(wv_ref)
        z_ref[...] = jnp.zeros_like(z_ref)
    bm = b_ref[...]
    kb = k_ref[0]
    qb = q_ref[0]
    vb = v_ref[0]
    parts = []
    zparts = []
    for h in range(H):
        kh = kb[:, h * DH:(h + 1) * DH]
        qh = qb[:, h * DH:(h + 1) * DH]
        vh = vb[:, h * DH:(h + 1) * DH]
        s = jax.lax.dot_general(kh, qh, (((1,), (1,)), ((), ())),
                                preferred_element_type=jnp.float32)
        e = jnp.exp(jnp.clip(s / 8.0, -5.0, 5.0)) * bm
        parts.append(jax.lax.dot_general(e, vh, (((0,), (0,)), ((), ())),
                                         preferred_element_type=jnp.float32))
        zparts.append(jnp.sum(e, axis=0).reshape(MB, 1))
    wv_ref[...] += jnp.concatenate(parts, axis=1)
    z_ref[...] += jnp.concatenate(zparts, axis=1)


def _attention(q, k, v, bmask, s):
    g = N // MB
    return pl.pallas_call(
        _attn_kernel,
        grid=(g, g),
        in_specs=[
            pl.BlockSpec((1, MB, 512), lambda j, i: (s, i, 0)),
            pl.BlockSpec((1, MB, 512), lambda j, i: (s, j, 0)),
            pl.BlockSpec((1, MB, 512), lambda j, i: (s, i, 0)),
            pl.BlockSpec((MB, MB), lambda j, i: (i, j)),
        ],
        out_specs=[pl.BlockSpec((MB, 512), lambda j, i: (j, 0)),
                   pl.BlockSpec((MB, H), lambda j, i: (j, 0))],
        out_shape=[jax.ShapeDtypeStruct((N, 512), jnp.float32),
                   jax.ShapeDtypeStruct((N, H), jnp.float32)],
        compiler_params=pltpu.CompilerParams(
            dimension_semantics=("parallel", "arbitrary")),
    )(k, q, v, bmask)


def _combine_kernel(h_ref, wv1_ref, wv2_ref, wv3_ref, z1_ref, z2_ref, z3_ref,
                    w1_ref, b1_ref, w2_ref, b2_ref, wo_ref, r1_ref, st_ref):
    hb = h_ref[...]
    dn = (((1,), (1,)), ((), ()))
    g1 = jnp.maximum(jax.lax.dot_general(hb, w1_ref[...], dn,
                                         preferred_element_type=jnp.float32)
                     + b1_ref[0, :][None, :], 0.0)
    lo = jax.lax.dot_general(g1, w2_ref[...], dn,
                             preferred_element_type=jnp.float32) \
        + b2_ref[0, :][None, :]
    lo = lo - jnp.max(lo, axis=1, keepdims=True)
    el = jnp.exp(lo)
    sw = el / jnp.sum(el, axis=1, keepdims=True)
    wvs = [wv1_ref[...], wv2_ref[...], wv3_ref[...]]
    zs = [z1_ref[...], z2_ref[...], z3_ref[...]]
    parts = []
    for h in range(H):
        acc = None
        for s in range(NS):
            o = wvs[s][:, h * DH:(h + 1) * DH] / (zs[s][:, h:h + 1] + 1e-6)
            t = sw[:, s:s + 1] * o
            acc = t if acc is None else acc + t
        parts.append(acc)
    attn = jnp.concatenate(parts, axis=1)
    r1 = hb + jax.lax.dot_general(attn, wo_ref[...], dn,
                                  preferred_element_type=jnp.float32)
    r1_ref[...] = r1
    @pl.when(pl.program_id(0) == 0)
    def _():
        st_ref[...] = jnp.zeros_like(st_ref)
    st_ref[...] += jnp.concatenate(
        [jnp.sum(r1, axis=0).reshape(1, 512),
         jnp.sum(r1 * r1, axis=0).reshape(1, 512)], axis=0)


def _combine(h, wv, z, w1, b1, w2, b2, wo):
    full = lambda r, c: pl.BlockSpec((r, c), lambda i: (0, 0))
    blk = lambda r, c: pl.BlockSpec((r, c), lambda i: (i, 0))
    return pl.pallas_call(
        _combine_kernel,
        grid=(N // RB,),
        in_specs=[blk(RB, 512), blk(RB, 512), blk(RB, 512), blk(RB, 512),
                  blk(RB, H), blk(RB, H), blk(RB, H),
                  full(16, 512), full(1, 16), full(3, 16), full(1, 3),
                  full(512, 512)],
        out_specs=[blk(RB, 512), full(2, 512)],
        out_shape=[jax.ShapeDtypeStruct((N, 512), jnp.float32),
                   jax.ShapeDtypeStruct((2, 512), jnp.float32)],
    )(h, wv[0], wv[1], wv[2], z[0], z[1], z[2], w1, b1, w2, b2, wo)


def _ffn_kernel(x_ref, st_ref, g_ref, b_ref, w1_ref, b1_ref, w2_ref, b2_ref,
                r2_ref, st2_ref):
    x = x_ref[...]
    m = st_ref[0, :][None, :] / N
    v = st_ref[1, :][None, :] / N - m * m
    n1 = (x - m) / jnp.sqrt(v + 1e-5) * g_ref[0, :][None, :] \
        + b_ref[0, :][None, :]
    dn = (((1,), (1,)), ((), ()))
    hid = jnp.maximum(jax.lax.dot_general(n1, w1_ref[...], dn,
                                          preferred_element_type=jnp.float32)
                      + b1_ref[0, :][None, :], 0.0)
    f = jax.lax.dot_general(hid, w2_ref[...], dn,
                            preferred_element_type=jnp.float32) \
        + b2_ref[0, :][None, :]
    r2 = n1 + f
    r2_ref[...] = r2
    @pl.when(pl.program_id(0) == 0)
    def _():
        st2_ref[...] = jnp.zeros_like(st2_ref)
    st2_ref[...] += jnp.concatenate(
        [jnp.sum(r2, axis=0).reshape(1, 512),
         jnp.sum(r2 * r2, axis=0).reshape(1, 512)], axis=0)


def _ffn(x, st, g, b, w1, b1, w2, b2):
    full = lambda r, c: pl.BlockSpec((r, c), lambda i: (0, 0))
    blk = pl.BlockSpec((RB, 512), lambda i: (i, 0))
    return pl.pallas_call(
        _ffn_kernel,
        grid=(N // RB,),
        in_specs=[blk, full(2, 512), full(1, 512), full(1, 512),
                  full(1024, 512), full(1, 1024), full(512, 1024),
                  full(1, 512)],
        out_specs=[blk, full(2, 512)],
        out_shape=[jax.ShapeDtypeStruct((N, 512), jnp.float32),
                   jax.ShapeDtypeStruct((2, 512), jnp.float32)],
    )(x, st, g, b, w1, b1, w2, b2)


def _bn_kernel(x_ref, st_ref, g_ref, b_ref, o_ref):
    x = x_ref[...]
    m = st_ref[0, :][None, :] / N
    v = st_ref[1, :][None, :] / N - m * m
    o_ref[...] = (x - m) / jnp.sqrt(v + 1e-5) * g_ref[0, :][None, :] \
        + b_ref[0, :][None, :]


def _bn_final(x, st, g, b):
    full = lambda r, c: pl.BlockSpec((r, c), lambda i: (0, 0))
    blk = pl.BlockSpec((RB, 512), lambda i: (i, 0))
    return pl.pallas_call(
        _bn_kernel,
        grid=(N // RB,),
        in_specs=[blk, full(2, 512), full(1, 512), full(1, 512)],
        out_specs=blk,
        out_shape=jax.ShapeDtypeStruct((N, 512), jnp.float32),
    )(x, st, g, b)


def kernel(h, edge_index, Wq, Wk, Wv, swg_W1, swg_b1, swg_W2, swg_b2, WO,
           bn1_g, bn1_b, ffn_W1, ffn_b1, ffn_W2, ffn_b2, bn2_g, bn2_b):
    src = edge_index[0]
    dst = edge_index[1]
    adj = jnp.zeros((N, N), jnp.float32).at[
        jnp.concatenate([src, dst]), jnp.concatenate([dst, src])].add(1.0)

    d = _rowsum(adj)
    m1 = _normalize(adj, d)
    m2 = _matmul(m1, m1)
    m4 = _matmul(m2, m2)

    masks = [_topk_mask(m1), _topk_mask(m2), _topk_mask(m4)]

    q, k, v = _qkv(h, Wq, Wk, Wv)

    wvs, zs = [], []
    for s in range(NS):
        wv_s, z_s = _attention(q, k, v, masks[s], s)
        wvs.append(wv_s)
        zs.append(z_s)

    r1, st1 = _combine(h, wvs, zs, swg_W1, swg_b1.reshape(1, 16),
                       swg_W2, swg_b2.reshape(1, 3), WO)
    r2, st2 = _ffn(r1, st1, bn1_g.reshape(1, 512), bn1_b.reshape(1, 512),
                   ffn_W1, ffn_b1.reshape(1, 1024), ffn_W2,
                   ffn_b2.reshape(1, 512))
    return _bn_final(r2, st2, bn2_g.reshape(1, 512), bn2_b.reshape(1, 512))


# speedup vs baseline: 14.4211x; 1.2163x over previous
import jax
import jax.numpy as jnp
from jax.experimental import pallas as pl
from jax.experimental.pallas import tpu as pltpu

N = 4096
H = 8
DH = 64
NS = 3
TOPK = 20
RB = 256
MB = 512


def _rowsum_kernel(a_ref, d_ref):
    d_ref[...] = jnp.sum(a_ref[...], axis=1).reshape(1, RB)


def _rowsum(a):
    return pl.pallas_call(
        _rowsum_kernel,
        grid=(N // RB,),
        in_specs=[pl.BlockSpec((RB, N), lambda i: (i, 0))],
        out_specs=pl.BlockSpec((1, RB), lambda i: (0, i)),
        out_shape=jax.ShapeDtypeStruct((1, N), jnp.float32),
    )(a)


def _norm_kernel(a_ref, d_ref, m_ref):
    i = pl.program_id(0)
    d = d_ref[0, :]
    dinv = jnp.where(d > 0, d ** -0.5, 0.0)
    rows = d_ref[0, pl.ds(i * RB, RB)]
    rinv = jnp.where(rows > 0, rows ** -0.5, 0.0)
    m_ref[...] = a_ref[...] * rinv[:, None] * dinv[None, :]


def _normalize(a, d):
    return pl.pallas_call(
        _norm_kernel,
        grid=(N // RB,),
        in_specs=[pl.BlockSpec((RB, N), lambda i: (i, 0)),
                  pl.BlockSpec((1, N), lambda i: (0, 0))],
        out_specs=pl.BlockSpec((RB, N), lambda i: (i, 0)),
        out_shape=jax.ShapeDtypeStruct((N, N), jnp.float32),
    )(a, d)


def _mm_kernel(a_ref, b_ref, o_ref):
    o_ref[...] = jnp.dot(a_ref[...], b_ref[...],
                         preferred_element_type=jnp.float32)


def _matmul(a, b):
    g = N // MB
    return pl.pallas_call(
        _mm_kernel,
        grid=(g, g),
        in_specs=[pl.BlockSpec((MB, N), lambda i, j: (i, 0)),
                  pl.BlockSpec((N, MB), lambda i, j: (0, j))],
        out_specs=pl.BlockSpec((MB, MB), lambda i, j: (i, j)),
        out_shape=jax.ShapeDtypeStruct((N, N), jnp.float32),
        compiler_params=pltpu.CompilerParams(
            dimension_semantics=("parallel", "parallel")),
    )(a, b)


def _topk_kernel(r_ref, m_ref):
    i = pl.program_id(0)
    x = r_ref[...]
    col = jax.lax.broadcasted_iota(jnp.int32, (RB, N), 1)
    rowg = jax.lax.broadcasted_iota(jnp.int32, (RB, N), 0) + i * RB
    x = jnp.where(col == rowg, 0.0, x)
    mask = jnp.zeros_like(x)
    for _ in range(TOPK):
        mx = jnp.max(x, axis=1, keepdims=True)
        cand = jnp.where(x == mx, col, N)
        jmin = jnp.min(cand, axis=1, keepdims=True)
        sel = col == jmin
        mask = jnp.where(sel, 1.0, mask)
        x = jnp.where(sel, -1.0, x)
    m_ref[...] = mask


def _topk_mask(r):
    return pl.pallas_call(
        _topk_kernel,
        grid=(N // RB,),
        in_specs=[pl.BlockSpec((RB, N), lambda i: (i, 0))],
        out_specs=pl.BlockSpec((RB, N), lambda i: (i, 0)),
        out_shape=jax.ShapeDtypeStruct((N, N), jnp.float32),
    )(r)


def _qkv_kernel(h_ref, wq_ref, wk_ref, wv_ref, q_ref, k_ref, v_ref):
    hb = h_ref[...]
    dn = (((1,), (1,)), ((), ()))
    for s in range(NS):
        q_ref[s] = jax.lax.dot_general(hb, wq_ref[s], dn,
                                       preferred_element_type=jnp.float32)
        k_ref[s] = jax.lax.dot_general(hb, wk_ref[s], dn,
                                       preferred_element_type=jnp.float32)
        v_ref[s] = jax.lax.dot_general(hb, wv_ref[s], dn,
                                       preferred_element_type=jnp.float32)


def _qkv(h, wq, wk, wv):
    wspec = pl.BlockSpec((NS, 512, 512), lambda i: (0, 0, 0))
    ospec = pl.BlockSpec((NS, RB, 512), lambda i: (0, i, 0))
    osh = jax.ShapeDtypeStruct((NS, N, 512), jnp.float32)
    return pl.pallas_call(
        _qkv_kernel,
        grid=(N // RB,),
        in_specs=[pl.BlockSpec((RB, 512), lambda i: (i, 0)), wspec, wspec,
                  wspec],
        out_specs=[ospec, ospec, ospec],
        out_shape=[osh, osh, osh],
    )(h, wq, wk, wv)


def _attn_kernel(k_ref, q_ref, v_ref, b_ref, wv_ref, z_ref):
    @pl.when(pl.program_id(1) == 0)
    def _():
        wv_ref[...] = jnp.zeros_like(wv_ref)
        z_ref[...] = jnp.zeros_like(z_ref)
    bm = b_ref[...]
    kb = k_ref[0]
    qb = q_ref[0]
    vb = v_ref[0]
    parts = []
    zparts = []
    for h in range(H):
        kh = kb[:, h * DH:(h + 1) * DH]
        qh = qb[:, h * DH:(h + 1) * DH]
        vh = vb[:, h * DH:(h + 1) * DH]
        s = jax.lax.dot_general(kh, qh, (((1,), (1,)), ((), ())),
                                preferred_element_type=jnp.float32)
        e = jnp.exp(jnp.clip(s / 8.0, -5.0, 5.0)) * bm
        parts.append(jax.lax.dot_general(e, vh, (((0,), (0,)), ((), ())),
                                         preferred_element_type=jnp.float32))
        zparts.append(jnp.sum(e, axis=0).reshape(MB, 1))
    wv_ref[...] += jnp.concatenate(parts, axis=1)
    z_ref[...] += jnp.concatenate(zparts, axis=1)


def _attention(q, k, v, bmask, s):
    g = N // MB
    return pl.pallas_call(
        _attn_kernel,
        grid=(g, g),
        in_specs=[
            pl.BlockSpec((1, MB, 512), lambda j, i: (s, i, 0)),
            pl.BlockSpec((1, MB, 512), lambda j, i: (s, j, 0)),
            pl.BlockSpec((1, MB, 512), lambda j, i: (s, i, 0)),
            pl.BlockSpec((MB, MB), lambda j, i: (i, j)),
        ],
        out_specs=[pl.BlockSpec((MB, 512), lambda j, i: (j, 0)),
                   pl.BlockSpec((MB, H), lambda j, i: (j, 0))],
        out_shape=[jax.ShapeDtypeStruct((N, 512), jnp.float32),
                   jax.ShapeDtypeStruct((N, H), jnp.float32)],
        compiler_params=pltpu.CompilerParams(
            dimension_semantics=("parallel", "arbitrary")),
    )(k, q, v, bmask)


def _combine_kernel(h_ref, wv1_ref, wv2_ref, wv3_ref, z1_ref, z2_ref, z3_ref,
                    w1_ref, b1_ref, w2_ref, b2_ref, wo_ref, r1_ref, st_ref):
    hb = h_ref[...]
    dn = (((1,), (1,)), ((), ()))
    g1 = jnp.maximum(jax.lax.dot_general(hb, w1_ref[...], dn,
                                         preferred_element_type=jnp.float32)
                     + b1_ref[0, :][None, :], 0.0)
    lo = jax.lax.dot_general(g1, w2_ref[...], dn,
                             preferred_element_type=jnp.float32) \
        + b2_ref[0, :][None, :]
    lo = lo - jnp.max(lo, axis=1, keepdims=True)
    el = jnp.exp(lo)
    sw = el / jnp.sum(el, axis=1, keepdims=True)
    wvs = [wv1_ref[...], wv2_ref[...], wv3_ref[...]]
    zs = [z1_ref[...], z2_ref[...], z3_ref[...]]
    parts = []
    for h in range(H):
        acc = None
        for s in range(NS):
            o = wvs[s][:, h * DH:(h + 1) * DH] / (zs[s][:, h:h + 1] + 1e-6)
            t = sw[:, s:s + 1] * o
            acc = t if acc is None else acc + t
        parts.append(acc)
    attn = jnp.concatenate(parts, axis=1)
    r1 = hb + jax.lax.dot_general(attn, wo_ref[...], dn,
                                  preferred_element_type=jnp.float32)
    r1_ref[...] = r1
    @pl.when(pl.program_id(0) == 0)
    def _():
        st_ref[...] = jnp.zeros_like(st_ref)
    st_ref[...] += jnp.concatenate(
        [jnp.sum(r1, axis=0).reshape(1, 512),
         jnp.sum(r1 * r1, axis=0).reshape(1, 512)], axis=0)


def _combine(h, wv, z, w1, b1, w2, b2, wo):
    full = lambda r, c: pl.BlockSpec((r, c), lambda i: (0, 0))
    blk = lambda r, c: pl.BlockSpec((r, c), lambda i: (i, 0))
    return pl.pallas_call(
        _combine_kernel,
        grid=(N // RB,),
        in_specs=[blk(RB, 512), blk(RB, 512), blk(RB, 512), blk(RB, 512),
                  blk(RB, H), blk(RB, H), blk(RB, H),
                  full(16, 512), full(1, 16), full(3, 16), full(1, 3),
                  full(512, 512)],
        out_specs=[blk(RB, 512), full(2, 512)],
        out_shape=[jax.ShapeDtypeStruct((N, 512), jnp.float32),
                   jax.ShapeDtypeStruct((2, 512), jnp.float32)],
    )(h, wv[0], wv[1], wv[2], z[0], z[1], z[2], w1, b1, w2, b2, wo)


def _ffn_kernel(x_ref, st_ref, g_ref, b_ref, w1_ref, b1_ref, w2_ref, b2_ref,
                r2_ref, st2_ref):
    x = x_ref[...]
    m = st_ref[0, :][None, :] / N
    v = st_ref[1, :][None, :] / N - m * m
    n1 = (x - m) / jnp.sqrt(v + 1e-5) * g_ref[0, :][None, :] \
        + b_ref[0, :][None, :]
    dn = (((1,), (1,)), ((), ()))
    hid = jnp.maximum(jax.lax.dot_general(n1, w1_ref[...], dn,
                                          preferred_element_type=jnp.float32)
                      + b1_ref[0, :][None, :], 0.0)
    f = jax.lax.dot_general(hid, w2_ref[...], dn,
                            preferred_element_type=jnp.float32) \
        + b2_ref[0, :][None, :]
    r2 = n1 + f
    r2_ref[...] = r2
    @pl.when(pl.program_id(0) == 0)
    def _():
        st2_ref[...] = jnp.zeros_like(st2_ref)
    st2_ref[...] += jnp.concatenate(
        [jnp.sum(r2, axis=0).reshape(1, 512),
         jnp.sum(r2 * r2, axis=0).reshape(1, 512)], axis=0)


def _ffn(x, st, g, b, w1, b1, w2, b2):
    full = lambda r, c: pl.BlockSpec((r, c), lambda i: (0, 0))
    blk = pl.BlockSpec((RB, 512), lambda i: (i, 0))
    return pl.pallas_call(
        _ffn_kernel,
        grid=(N // RB,),
        in_specs=[blk, full(2, 512), full(1, 512), full(1, 512),
                  full(1024, 512), full(1, 1024), full(512, 1024),
                  full(1, 512)],
        out_specs=[blk, full(2, 512)],
        out_shape=[jax.ShapeDtypeStruct((N, 512), jnp.float32),
                   jax.ShapeDtypeStruct((2, 512), jnp.float32)],
    )(x, st, g, b, w1, b1, w2, b2)


def _bn_kernel(x_ref, st_ref, g_ref, b_ref, o_ref):
    x = x_ref[...]
    m = st_ref[0, :][None, :] / N
    v = st_ref[1, :][None, :] / N - m * m
    o_ref[...] = (x - m) / jnp.sqrt(v + 1e-5) * g_ref[0, :][None, :] \
        + b_ref[0, :][None, :]


def _bn_final(x, st, g, b):
    full = lambda r, c: pl.BlockSpec((r, c), lambda i: (0, 0))
    blk = pl.BlockSpec((RB, 512), lambda i: (i, 0))
    return pl.pallas_call(
        _bn_kernel,
        grid=(N // RB,),
        in_specs=[blk, full(2, 512), full(1, 512), full(1, 512)],
        out_specs=blk,
        out_shape=jax.ShapeDtypeStruct((N, 512), jnp.float32),
    )(x, st, g, b)


def kernel(h, edge_index, Wq, Wk, Wv, swg_W1, swg_b1, swg_W2, swg_b2, WO,
           bn1_g, bn1_b, ffn_W1, ffn_b1, ffn_W2, ffn_b2, bn2_g, bn2_b):
    src = edge_index[0]
    dst = edge_index[1]
    adj = jnp.zeros((N, N), jnp.float32).at[
        jnp.concatenate([src, dst]), jnp.concatenate([dst, src])].add(1.0)

    d = _rowsum(adj)
    m1 = _normalize(adj, d)
    m2 = _matmul(m1, m1)
    m4 = _matmul(m2, m2)

    masks = [_topk_mask(m1), _topk_mask(m2), _topk_mask(m4)]

    q, k, v = _qkv(h, Wq, Wk, Wv)

    wvs, zs = [], []
    for s in range(NS):
        wv_s, z_s = _attention(q, k, v, masks[s], s)
        wvs.append(wv_s)
        zs.append(z_s)

    r1, st1 = _combine(h, wvs, zs, swg_W1, swg_b1.reshape(1, 16),
                       swg_W2, swg_b2.reshape(1, 3), WO)
    r2, st2 = _ffn(r1, st1, bn1_g.reshape(1, 512), bn1_b.reshape(1, 512),
                   ffn_W1, ffn_b1.reshape(1, 1024), ffn_W2,
                   ffn_b2.reshape(1, 512))
    return _bn_final(r2, st2, bn2_g.reshape(1, 512), bn2_b.reshape(1, 512))


# speedup vs baseline: 15.7506x; 1.0922x over previous
import jax
import jax.numpy as jnp
from jax.experimental import pallas as pl
from jax.experimental.pallas import tpu as pltpu

N = 4096
H = 8
DH = 64
NS = 3
TOPK = 20
RB = 256
MB = 512


def _rowsum_kernel(a_ref, d_ref):
    d_ref[...] = jnp.sum(a_ref[...], axis=1).reshape(1, RB)


def _rowsum(a):
    return pl.pallas_call(
        _rowsum_kernel,
        grid=(N // RB,),
        in_specs=[pl.BlockSpec((RB, N), lambda i: (i, 0))],
        out_specs=pl.BlockSpec((1, RB), lambda i: (0, i)),
        out_shape=jax.ShapeDtypeStruct((1, N), jnp.float32),
    )(a)


def _norm_kernel(a_ref, d_ref, m_ref):
    i = pl.program_id(0)
    d = d_ref[0, :]
    dinv = jnp.where(d > 0, d ** -0.5, 0.0)
    rows = d_ref[0, pl.ds(i * RB, RB)]
    rinv = jnp.where(rows > 0, rows ** -0.5, 0.0)
    m_ref[...] = a_ref[...] * rinv[:, None] * dinv[None, :]


def _normalize(a, d):
    return pl.pallas_call(
        _norm_kernel,
        grid=(N // RB,),
        in_specs=[pl.BlockSpec((RB, N), lambda i: (i, 0)),
                  pl.BlockSpec((1, N), lambda i: (0, 0))],
        out_specs=pl.BlockSpec((RB, N), lambda i: (i, 0)),
        out_shape=jax.ShapeDtypeStruct((N, N), jnp.float32),
    )(a, d)


def _mm_kernel(a_ref, b_ref, o_ref):
    o_ref[...] = jnp.dot(a_ref[...], b_ref[...],
                         preferred_element_type=jnp.float32)


def _matmul(a, b):
    g = N // MB
    return pl.pallas_call(
        _mm_kernel,
        grid=(g, g),
        in_specs=[pl.BlockSpec((MB, N), lambda i, j: (i, 0)),
                  pl.BlockSpec((N, MB), lambda i, j: (0, j))],
        out_specs=pl.BlockSpec((MB, MB), lambda i, j: (i, j)),
        out_shape=jax.ShapeDtypeStruct((N, N), jnp.float32),
        compiler_params=pltpu.CompilerParams(
            dimension_semantics=("parallel", "parallel")),
    )(a, b)


def _topk_kernel(r_ref, m_ref):
    i = pl.program_id(0)
    x = r_ref[...]
    col = jax.lax.broadcasted_iota(jnp.int32, (RB, N), 1)
    rowg = jax.lax.broadcasted_iota(jnp.int32, (RB, N), 0) + i * RB
    x = jnp.where(col == rowg, 0.0, x)
    for _ in range(TOPK):
        mx = jnp.max(x, axis=1, keepdims=True)
        cand = jnp.where(x == mx, col, N)
        jmin = jnp.min(cand, axis=1, keepdims=True)
        x = jnp.where(col == jmin, -1.0, x)
    m_ref[...] = jnp.where(x < 0.0, 1.0, 0.0)


def _topk_mask(r):
    return pl.pallas_call(
        _topk_kernel,
        grid=(N // RB,),
        in_specs=[pl.BlockSpec((RB, N), lambda i: (i, 0))],
        out_specs=pl.BlockSpec((RB, N), lambda i: (i, 0)),
        out_shape=jax.ShapeDtypeStruct((N, N), jnp.float32),
    )(r)


def _qkv_kernel(h_ref, wq_ref, wk_ref, wv_ref, q_ref, k_ref, v_ref):
    hb = h_ref[...]
    dn = (((1,), (1,)), ((), ()))
    for s in range(NS):
        q_ref[s] = jax.lax.dot_general(hb, wq_ref[s], dn,
                                       preferred_element_type=jnp.float32)
        k_ref[s] = jax.lax.dot_general(hb, wk_ref[s], dn,
                                       preferred_element_type=jnp.float32)
        v_ref[s] = jax.lax.dot_general(hb, wv_ref[s], dn,
                                       preferred_element_type=jnp.float32)


def _qkv(h, wq, wk, wv):
    wspec = pl.BlockSpec((NS, 512, 512), lambda i: (0, 0, 0))
    ospec = pl.BlockSpec((NS, RB, 512), lambda i: (0, i, 0))
    osh = jax.ShapeDtypeStruct((NS, N, 512), jnp.float32)
    return pl.pallas_call(
        _qkv_kernel,
        grid=(N // RB,),
        in_specs=[pl.BlockSpec((RB, 512), lambda i: (i, 0)), wspec, wspec,
                  wspec],
        out_specs=[ospec, ospec, ospec],
        out_shape=[osh, osh, osh],
    )(h, wq, wk, wv)


def _attn_kernel(k_ref, q_ref, v_ref, b_ref, wv_ref, z_ref):
    @pl.when(pl.program_id(1) == 0)
    def _():
        wv_ref[...] = jnp.zeros_like(wv_ref)
        z_ref[...] = jnp.zeros_like(z_ref)
    bm = b_ref[...]
    kb = k_ref[0]
    qb = q_ref[0]
    vb = v_ref[0]
    parts = []
    zparts = []
    for h in range(H):
        kh = kb[:, h * DH:(h + 1) * DH]
        qh = qb[:, h * DH:(h + 1) * DH]
        vh = vb[:, h * DH:(h + 1) * DH]
        s = jax.lax.dot_general(kh, qh, (((1,), (1,)), ((), ())),
                                preferred_element_type=jnp.float32)
        e = jnp.exp(jnp.clip(s / 8.0, -5.0, 5.0)) * bm
        parts.append(jax.lax.dot_general(e, vh, (((0,), (0,)), ((), ())),
                                         preferred_element_type=jnp.float32))
        zparts.append(jnp.sum(e, axis=0).reshape(MB, 1))
    wv_ref[...] += jnp.concatenate(parts, axis=1)
    z_ref[...] += jnp.concatenate(zparts, axis=1)


def _attention(q, k, v, bmask, s):
    g = N // MB
    return pl.pallas_call(
        _attn_kernel,
        grid=(g, g),
        in_specs=[
            pl.BlockSpec((1, MB, 512), lambda j, i: (s, i, 0)),
            pl.BlockSpec((1, MB, 512), lambda j, i: (s, j, 0)),
            pl.BlockSpec((1, MB, 512), lambda j, i: (s, i, 0)),
            pl.BlockSpec((MB, MB), lambda j, i: (i, j)),
        ],
        out_specs=[pl.BlockSpec((MB, 512), lambda j, i: (j, 0)),
                   pl.BlockSpec((MB, H), lambda j, i: (j, 0))],
        out_shape=[jax.ShapeDtypeStruct((N, 512), jnp.float32),
                   jax.ShapeDtypeStruct((N, H), jnp.float32)],
        compiler_params=pltpu.CompilerParams(
            dimension_semantics=("parallel", "arbitrary")),
    )(k, q, v, bmask)


def _combine_kernel(h_ref, wv1_ref, wv2_ref, wv3_ref, z1_ref, z2_ref, z3_ref,
                    w1_ref, b1_ref, w2_ref, b2_ref, wo_ref, r1_ref, st_ref):
    hb = h_ref[...]
    dn = (((1,), (1,)), ((), ()))
    g1 = jnp.maximum(jax.lax.dot_general(hb, w1_ref[...], dn,
                                         preferred_element_type=jnp.float32)
                     + b1_ref[0, :][None, :], 0.0)
    lo = jax.lax.dot_general(g1, w2_ref[...], dn,
                             preferred_element_type=jnp.float32) \
        + b2_ref[0, :][None, :]
    lo = lo - jnp.max(lo, axis=1, keepdims=True)
    el = jnp.exp(lo)
    sw = el / jnp.sum(el, axis=1, keepdims=True)
    wvs = [wv1_ref[...], wv2_ref[...], wv3_ref[...]]
    zs = [z1_ref[...], z2_ref[...], z3_ref[...]]
    parts = []
    for h in range(H):
        acc = None
        for s in range(NS):
            o = wvs[s][:, h * DH:(h + 1) * DH] / (zs[s][:, h:h + 1] + 1e-6)
            t = sw[:, s:s + 1] * o
            acc = t if acc is None else acc + t
        parts.append(acc)
    attn = jnp.concatenate(parts, axis=1)
    r1 = hb + jax.lax.dot_general(attn, wo_ref[...], dn,
                                  preferred_element_type=jnp.float32)
    r1_ref[...] = r1
    @pl.when(pl.program_id(0) == 0)
    def _():
        st_ref[...] = jnp.zeros_like(st_ref)
    st_ref[...] += jnp.concatenate(
        [jnp.sum(r1, axis=0).reshape(1, 512),
         jnp.sum(r1 * r1, axis=0).reshape(1, 512)], axis=0)


def _combine(h, wv, z, w1, b1, w2, b2, wo):
    full = lambda r, c: pl.BlockSpec((r, c), lambda i: (0, 0))
    blk = lambda r, c: pl.BlockSpec((r, c), lambda i: (i, 0))
    return pl.pallas_call(
        _combine_kernel,
        grid=(N // RB,),
        in_specs=[blk(RB, 512), blk(RB, 512), blk(RB, 512), blk(RB, 512),
                  blk(RB, H), blk(RB, H), blk(RB, H),
                  full(16, 512), full(1, 16), full(3, 16), full(1, 3),
                  full(512, 512)],
        out_specs=[blk(RB, 512), full(2, 512)],
        out_shape=[jax.ShapeDtypeStruct((N, 512), jnp.float32),
                   jax.ShapeDtypeStruct((2, 512), jnp.float32)],
    )(h, wv[0], wv[1], wv[2], z[0], z[1], z[2], w1, b1, w2, b2, wo)


def _ffn_kernel(x_ref, st_ref, g_ref, b_ref, w1_ref, b1_ref, w2_ref, b2_ref,
                r2_ref, st2_ref):
    x = x_ref[...]
    m = st_ref[0, :][None, :] / N
    v = st_ref[1, :][None, :] / N - m * m
    n1 = (x - m) / jnp.sqrt(v + 1e-5) * g_ref[0, :][None, :] \
        + b_ref[0, :][None, :]
    dn = (((1,), (1,)), ((), ()))
    hid = jnp.maximum(jax.lax.dot_general(n1, w1_ref[...], dn,
                                          preferred_element_type=jnp.float32)
                      + b1_ref[0, :][None, :], 0.0)
    f = jax.lax.dot_general(hid, w2_ref[...], dn,
                            preferred_element_type=jnp.float32) \
        + b2_ref[0, :][None, :]
    r2 = n1 + f
    r2_ref[...] = r2
    @pl.when(pl.program_id(0) == 0)
    def _():
        st2_ref[...] = jnp.zeros_like(st2_ref)
    st2_ref[...] += jnp.concatenate(
        [jnp.sum(r2, axis=0).reshape(1, 512),
         jnp.sum(r2 * r2, axis=0).reshape(1, 512)], axis=0)


def _ffn(x, st, g, b, w1, b1, w2, b2):
    full = lambda r, c: pl.BlockSpec((r, c), lambda i: (0, 0))
    blk = pl.BlockSpec((RB, 512), lambda i: (i, 0))
    return pl.pallas_call(
        _ffn_kernel,
        grid=(N // RB,),
        in_specs=[blk, full(2, 512), full(1, 512), full(1, 512),
                  full(1024, 512), full(1, 1024), full(512, 1024),
                  full(1, 512)],
        out_specs=[blk, full(2, 512)],
        out_shape=[jax.ShapeDtypeStruct((N, 512), jnp.float32),
                   jax.ShapeDtypeStruct((2, 512), jnp.float32)],
    )(x, st, g, b, w1, b1, w2, b2)


def _bn_kernel(x_ref, st_ref, g_ref, b_ref, o_ref):
    x = x_ref[...]
    m = st_ref[0, :][None, :] / N
    v = st_ref[1, :][None, :] / N - m * m
    o_ref[...] = (x - m) / jnp.sqrt(v + 1e-5) * g_ref[0, :][None, :] \
        + b_ref[0, :][None, :]


def _bn_final(x, st, g, b):
    full = lambda r, c: pl.BlockSpec((r, c), lambda i: (0, 0))
    blk = pl.BlockSpec((RB, 512), lambda i: (i, 0))
    return pl.pallas_call(
        _bn_kernel,
        grid=(N // RB,),
        in_specs=[blk, full(2, 512), full(1, 512), full(1, 512)],
        out_specs=blk,
        out_shape=jax.ShapeDtypeStruct((N, 512), jnp.float32),
    )(x, st, g, b)


def kernel(h, edge_index, Wq, Wk, Wv, swg_W1, swg_b1, swg_W2, swg_b2, WO,
           bn1_g, bn1_b, ffn_W1, ffn_b1, ffn_W2, ffn_b2, bn2_g, bn2_b):
    src = edge_index[0]
    dst = edge_index[1]
    adj = jnp.zeros((N, N), jnp.float32).at[
        jnp.concatenate([src, dst]), jnp.concatenate([dst, src])].add(1.0)

    d = _rowsum(adj)
    m1 = _normalize(adj, d)
    m2 = _matmul(m1, m1)
    m4 = _matmul(m2, m2)

    masks = [_topk_mask(m1), _topk_mask(m2), _topk_mask(m4)]

    q, k, v = _qkv(h, Wq, Wk, Wv)

    wvs, zs = [], []
    for s in range(NS):
        wv_s, z_s = _attention(q, k, v, masks[s], s)
        wvs.append(wv_s)
        zs.append(z_s)

    r1, st1 = _combine(h, wvs, zs, swg_W1, swg_b1.reshape(1, 16),
                       swg_W2, swg_b2.reshape(1, 3), WO)
    r2, st2 = _ffn(r1, st1, bn1_g.reshape(1, 512), bn1_b.reshape(1, 512),
                   ffn_W1, ffn_b1.reshape(1, 1024), ffn_W2,
                   ffn_b2.reshape(1, 512))
    return _bn_final(r2, st2, bn2_g.reshape(1, 512), bn2_b.reshape(1, 512))
